# Initial kernel scaffold; baseline (speedup 1.0000x reference)
import functools

import jax
import jax.numpy as jnp
from jax import lax
from jax.experimental import pallas as pl
from jax.experimental.pallas import tpu as pltpu
from jax.experimental.pallas import tpu_sc as plsc

VOCAB = 100000
EMBED_DIM = 64
B = 4096
T = 200
N = B * T

NC = 2
NS = 16
NW = NC * NS

PER_W = N // NW
CHUNK = 512
SUB = 128
N_SUB = CHUNK // SUB
STEPS = PER_W // CHUNK


def _body(idx_hbm, table_hbm, out_hbm, idx_v, rows_v, sem):
    wid = lax.axis_index("s") * NC + lax.axis_index("c")
    w_base = wid * PER_W

    def step(i, carry):
        base = w_base + i * CHUNK
        pltpu.sync_copy(idx_hbm.at[pl.ds(base, CHUNK)], idx_v)
        copies = []
        for j in range(N_SUB):
            copies.append(
                pltpu.async_copy(
                    table_hbm.at[idx_v.at[pl.ds(j * SUB, SUB)]],
                    rows_v.at[pl.ds(j * SUB, SUB)],
                    sem,
                )
            )
        for c in copies:
            c.wait()
        pltpu.sync_copy(rows_v, out_hbm.at[pl.ds(base, CHUNK)])
        return carry

    lax.fori_loop(0, STEPS, step, 0)


@jax.jit
def _embed(idx_flat, table):
    mesh = plsc.VectorSubcoreMesh(core_axis_name="c", subcore_axis_name="s")
    kern = pl.kernel(
        _body,
        out_type=jax.ShapeDtypeStruct((N, EMBED_DIM), jnp.float32),
        mesh=mesh,
        scratch_types=[
            pltpu.VMEM((CHUNK,), jnp.int32),
            pltpu.VMEM((CHUNK, EMBED_DIM), jnp.float32),
            pltpu.SemaphoreType.DMA,
        ],
    )
    return kern(idx_flat, table)


def kernel(action_tokens, table):
    idx_flat = action_tokens.reshape(-1).astype(jnp.int32)
    out = _embed(idx_flat, table)
    return out.reshape(B, T, EMBED_DIM)


# speedup vs baseline: 3.9513x; 3.9513x over previous
import functools

import jax
import jax.numpy as jnp
from jax import lax
from jax.experimental import pallas as pl
from jax.experimental.pallas import tpu as pltpu
from jax.experimental.pallas import tpu_sc as plsc

VOCAB = 100000
EMBED_DIM = 64
B = 4096
T = 200
N = B * T

NC = 2
NS = 16
NW = NC * NS

PER_W = N // NW
CHUNK = 512
SUB = 128
N_SUB = CHUNK // SUB
STEPS = PER_W // CHUNK


def _body(idx_hbm, table_hbm, out_hbm, idx_v, rows_v, sem):
    wid = lax.axis_index("s") * NC + lax.axis_index("c")
    w_base = wid * PER_W

    def step(i, carry):
        base = w_base + i * CHUNK
        pltpu.sync_copy(idx_hbm.at[pl.ds(base, CHUNK)], idx_v)
        copies = []
        for j in range(N_SUB):
            copies.append(
                pltpu.async_copy(
                    table_hbm.at[idx_v.at[pl.ds(j * SUB, SUB)]],
                    rows_v.at[pl.ds(j * SUB, SUB)],
                    sem,
                )
            )
        for c in copies:
            c.wait()
        pltpu.sync_copy(rows_v, out_hbm.at[pl.ds(base, CHUNK)])
        return carry

    lax.fori_loop(0, STEPS, step, 0)


@jax.jit
def _embed(idx_flat, table):
    mesh = plsc.VectorSubcoreMesh(core_axis_name="c", subcore_axis_name="s")
    kern = pl.kernel(
        _body,
        out_type=jax.ShapeDtypeStruct((N, EMBED_DIM), jnp.float32),
        mesh=mesh,
        scratch_types=[
            pltpu.VMEM((CHUNK,), jnp.int32),
            pltpu.VMEM((CHUNK, EMBED_DIM), jnp.float32),
            pltpu.SemaphoreType.DMA,
        ],
        compiler_params=pltpu.CompilerParams(use_tc_tiling_on_sc=False),
    )
    return kern(idx_flat, table)


def kernel(action_tokens, table):
    idx_flat = action_tokens.reshape(-1).astype(jnp.int32)
    out = _embed(idx_flat, table)
    return out.reshape(B, T, EMBED_DIM)


# speedup vs baseline: 4.2469x; 1.0748x over previous
import jax
import jax.numpy as jnp
from jax import lax
from jax.experimental import pallas as pl
from jax.experimental.pallas import tpu as pltpu
from jax.experimental.pallas import tpu_sc as plsc

VOCAB = 100000
EMBED_DIM = 64
B = 4096
T = 200
N = B * T

NC = 2
NS = 16
NW = NC * NS

PER_W = N // NW
CHUNK = 512
SUB = 128
N_SUB = CHUNK // SUB
STEPS = PER_W // CHUNK


def _fire_gather(table_hbm, idx_v, rows, sem, chunk_i):
    base = chunk_i * CHUNK
    for j in range(N_SUB):
        pltpu.async_copy(
            table_hbm.at[idx_v.at[pl.ds(base + j * SUB, SUB)]],
            rows.at[pl.ds(j * SUB, SUB)],
            sem,
        )


def _wait_rows(table_hbm, rows, sem):
    pltpu.make_async_copy(table_hbm.at[pl.ds(0, CHUNK)], rows, sem).wait()


def _fire_out(out_hbm, rows, sem, w_base, chunk_i):
    pltpu.async_copy(rows, out_hbm.at[pl.ds(w_base + chunk_i * CHUNK, CHUNK)], sem)


def _wait_out(out_hbm, rows, sem):
    pltpu.make_async_copy(rows, out_hbm.at[pl.ds(0, CHUNK)], sem).wait()


def _body(idx_hbm, table_hbm, out_hbm, idx_v, rows0, rows1, g0, g1, o0, o1):
    wid = lax.axis_index("s") * NC + lax.axis_index("c")
    w_base = wid * PER_W
    rows = (rows0, rows1)
    gsem = (g0, g1)
    osem = (o0, o1)

    pltpu.sync_copy(idx_hbm.at[pl.ds(w_base, PER_W)], idx_v)

    _fire_gather(table_hbm, idx_v, rows[0], gsem[0], 0)
    _wait_rows(table_hbm, rows[0], gsem[0])
    _fire_out(out_hbm, rows[0], osem[0], w_base, 0)
    _fire_gather(table_hbm, idx_v, rows[1], gsem[1], 1)

    def slot(i, b):
        _wait_rows(table_hbm, rows[b], gsem[b])
        _fire_out(out_hbm, rows[b], osem[b], w_base, i)
        _wait_out(out_hbm, rows[1 - b], osem[1 - b])
        _fire_gather(table_hbm, idx_v, rows[1 - b], gsem[1 - b], i + 1)

    def pair(g, carry):
        slot(1 + 2 * g, 1)
        slot(2 + 2 * g, 0)
        return carry

    lax.fori_loop(0, (STEPS - 2) // 2, pair, 0)

    bl = (STEPS - 1) % 2
    _wait_rows(table_hbm, rows[bl], gsem[bl])
    _fire_out(out_hbm, rows[bl], osem[bl], w_base, STEPS - 1)
    _wait_out(out_hbm, rows[1 - bl], osem[1 - bl])
    _wait_out(out_hbm, rows[bl], osem[bl])


@jax.jit
def _embed(idx_flat, table):
    mesh = plsc.VectorSubcoreMesh(core_axis_name="c", subcore_axis_name="s")
    kern = pl.kernel(
        _body,
        out_type=jax.ShapeDtypeStruct((N, EMBED_DIM), jnp.float32),
        mesh=mesh,
        scratch_types=[
            pltpu.VMEM((PER_W,), jnp.int32),
            pltpu.VMEM((CHUNK, EMBED_DIM), jnp.float32),
            pltpu.VMEM((CHUNK, EMBED_DIM), jnp.float32),
            pltpu.SemaphoreType.DMA,
            pltpu.SemaphoreType.DMA,
            pltpu.SemaphoreType.DMA,
            pltpu.SemaphoreType.DMA,
        ],
        compiler_params=pltpu.CompilerParams(use_tc_tiling_on_sc=False),
    )
    return kern(idx_flat, table)


def kernel(action_tokens, table):
    idx_flat = action_tokens.reshape(-1).astype(jnp.int32)
    out = _embed(idx_flat, table)
    return out.reshape(B, T, EMBED_DIM)


# speedup vs baseline: 4.2629x; 1.0038x over previous
import jax
import jax.numpy as jnp
from jax import lax
from jax.experimental import pallas as pl
from jax.experimental.pallas import tpu as pltpu
from jax.experimental.pallas import tpu_sc as plsc

VOCAB = 100000
EMBED_DIM = 64
B = 4096
T = 200
N = B * T

NC = 2
NS = 16
NW = NC * NS

PER_W = N // NW
CHUNK = 512
SUB = 512
N_SUB = CHUNK // SUB
STEPS = PER_W // CHUNK


def _fire_gather(table_hbm, idx_v, rows, sem, chunk_i):
    base = chunk_i * CHUNK
    for j in range(N_SUB):
        pltpu.async_copy(
            table_hbm.at[idx_v.at[pl.ds(base + j * SUB, SUB)]],
            rows.at[pl.ds(j * SUB, SUB)],
            sem,
        )


def _wait_rows(table_hbm, rows, sem):
    pltpu.make_async_copy(table_hbm.at[pl.ds(0, CHUNK)], rows, sem).wait()


def _fire_out(out_hbm, rows, sem, w_base, chunk_i):
    pltpu.async_copy(rows, out_hbm.at[pl.ds(w_base + chunk_i * CHUNK, CHUNK)], sem)


def _wait_out(out_hbm, rows, sem):
    pltpu.make_async_copy(rows, out_hbm.at[pl.ds(0, CHUNK)], sem).wait()


def _body(idx_hbm, table_hbm, out_hbm, idx_v, rows0, rows1, g0, g1, o0, o1):
    wid = lax.axis_index("s") * NC + lax.axis_index("c")
    w_base = wid * PER_W
    rows = (rows0, rows1)
    gsem = (g0, g1)
    osem = (o0, o1)

    pltpu.sync_copy(idx_hbm.at[pl.ds(w_base, PER_W)], idx_v)

    _fire_gather(table_hbm, idx_v, rows[0], gsem[0], 0)
    _wait_rows(table_hbm, rows[0], gsem[0])
    _fire_out(out_hbm, rows[0], osem[0], w_base, 0)
    _fire_gather(table_hbm, idx_v, rows[1], gsem[1], 1)

    def slot(i, b):
        _wait_rows(table_hbm, rows[b], gsem[b])
        _fire_out(out_hbm, rows[b], osem[b], w_base, i)
        _wait_out(out_hbm, rows[1 - b], osem[1 - b])
        _fire_gather(table_hbm, idx_v, rows[1 - b], gsem[1 - b], i + 1)

    def pair(g, carry):
        slot(1 + 2 * g, 1)
        slot(2 + 2 * g, 0)
        return carry

    lax.fori_loop(0, (STEPS - 2) // 2, pair, 0)

    bl = (STEPS - 1) % 2
    _wait_rows(table_hbm, rows[bl], gsem[bl])
    _fire_out(out_hbm, rows[bl], osem[bl], w_base, STEPS - 1)
    _wait_out(out_hbm, rows[1 - bl], osem[1 - bl])
    _wait_out(out_hbm, rows[bl], osem[bl])


@jax.jit
def _embed(idx_flat, table):
    mesh = plsc.VectorSubcoreMesh(core_axis_name="c", subcore_axis_name="s")
    kern = pl.kernel(
        _body,
        out_type=jax.ShapeDtypeStruct((N, EMBED_DIM), jnp.float32),
        mesh=mesh,
        scratch_types=[
            pltpu.VMEM((PER_W,), jnp.int32),
            pltpu.VMEM((CHUNK, EMBED_DIM), jnp.float32),
            pltpu.VMEM((CHUNK, EMBED_DIM), jnp.float32),
            pltpu.SemaphoreType.DMA,
            pltpu.SemaphoreType.DMA,
            pltpu.SemaphoreType.DMA,
            pltpu.SemaphoreType.DMA,
        ],
        compiler_params=pltpu.CompilerParams(use_tc_tiling_on_sc=False),
    )
    return kern(idx_flat, table)


def kernel(action_tokens, table):
    idx_flat = action_tokens.reshape(-1).astype(jnp.int32)
    out = _embed(idx_flat, table)
    return out.reshape(B, T, EMBED_DIM)
